# baseline (speedup 1.0000x reference)
import functools

import jax
import jax.numpy as jnp
from jax import lax
from jax.experimental import pallas as pl
from jax.experimental.pallas import tpu as pltpu
from jax.experimental.pallas import tpu_sc as plsc

VOCAB = 5053
EMBED_DIM = 32
SEQ = 100
BATCH = 16384

NUM_CORES = 2
NUM_SUBCORES = 16
NUM_WORKERS = NUM_CORES * NUM_SUBCORES

HALF_E = EMBED_DIM // 2
TC_PER_WORKER = (BATCH // 128) // (NUM_WORKERS // 2)
S_TILES_FULL = SEQ // 8
S_REM = SEQ - S_TILES_FULL * 8
UNITS_A = S_TILES_FULL * TC_PER_WORKER
VSTRIDE = 5056


def _encode(x_p, table_p):
    mesh = plsc.VectorSubcoreMesh(
        core_axis_name="c", subcore_axis_name="s",
        num_cores=NUM_CORES, num_subcores=NUM_SUBCORES,
    )

    @functools.partial(
        pl.kernel,
        out_type=jax.ShapeDtypeStruct((SEQ, EMBED_DIM, BATCH), jnp.float32),
        mesh=mesh,
        scratch_types=[
            pltpu.VMEM((HALF_E * VOCAB,), jnp.float32),
            pltpu.VMEM((2, 8, 128), jnp.int32),
            pltpu.VMEM((8, HALF_E, 128), jnp.float32),
            pltpu.SemaphoreType.DMA,
            pltpu.SemaphoreType.DMA((2,)),
            pltpu.SemaphoreType.DMA((8,)),
        ],
        compiler_params=pltpu.CompilerParams(needs_layout_passes=False),
    )
    def gather_kernel(x_hbm, tab_hbm, out_hbm,
                      tab_v, xbuf, obuf, sem_t, sem_x, sem_o):
        cid = lax.axis_index("c")
        sid = lax.axis_index("s")
        wid = sid * NUM_CORES + cid
        h = wid // (NUM_WORKERS // 2)
        grp = wid % (NUM_WORKERS // 2)
        e_base = h * HALF_E
        tc_base = grp * TC_PER_WORKER

        pltpu.async_copy(
            tab_hbm.at[pl.ds(e_base * VOCAB, HALF_E * VOCAB)], tab_v,
            sem_t).wait()

        def unit_coords(t):
            s_t = t // TC_PER_WORKER
            tc = tc_base + lax.rem(t, TC_PER_WORKER)
            return s_t * 8, tc * 128

        def start_xload(t, p):
            s0, b0 = unit_coords(t)
            pltpu.async_copy(
                x_hbm.at[pl.ds(s0, 8), pl.ds(b0, 128)], xbuf.at[p],
                sem_x.at[p])

        def wait_xload(t, p):
            s0, b0 = unit_coords(t)
            pltpu.make_async_copy(
                x_hbm.at[pl.ds(s0, 8), pl.ds(b0, 128)], xbuf.at[p],
                sem_x.at[p]).wait()

        def wait_oblock(s_r):
            pltpu.make_async_copy(
                obuf.at[s_r],
                out_hbm.at[0, pl.ds(e_base, HALF_E), pl.ds(0, 128)],
                sem_o.at[s_r]).wait()

        def compute_srow(p, s_r):
            vs = [xbuf[p, s_r, pl.ds(g * 16, 16)] for g in range(8)]

            @plsc.parallel_loop(0, HALF_E, unroll=2)
            def erow(e_r):
                base = jnp.full((16,), e_r * VOCAB, jnp.int32)
                for g in range(8):
                    vals = plsc.load_gather(tab_v, [vs[g] + base])
                    obuf[s_r, e_r, pl.ds(g * 16, 16)] = vals

        def store_srow(t, s_r):
            s0, b0 = unit_coords(t)
            pltpu.async_copy(
                obuf.at[s_r],
                out_hbm.at[s0 + s_r, pl.ds(e_base, HALF_E), pl.ds(b0, 128)],
                sem_o.at[s_r])

        start_xload(0, 0)

        def outer(tp, carry):
            for p in range(2):
                t = tp * 2 + p
                wait_xload(t, p)

                @pl.when(t < UNITS_A - 1)
                def _():
                    start_xload(t + 1, 1 - p)

                for s_r in range(8):
                    @pl.when(t >= 1)
                    def _():
                        wait_oblock(s_r)
                    compute_srow(p, s_r)
                    store_srow(t, s_r)
            return carry

        lax.fori_loop(0, UNITS_A // 2, outer, 0)

        for j in range(TC_PER_WORKER):
            b0 = (tc_base + j) * 128
            pltpu.async_copy(
                x_hbm.at[pl.ds(S_TILES_FULL * 8, S_REM), pl.ds(b0, 128)],
                xbuf.at[0, pl.ds(0, S_REM)], sem_x.at[0]).wait()
            for s_r in range(S_REM):
                wait_oblock(s_r)
                compute_srow(0, s_r)
                pltpu.async_copy(
                    obuf.at[s_r],
                    out_hbm.at[S_TILES_FULL * 8 + s_r,
                               pl.ds(e_base, HALF_E), pl.ds(b0, 128)],
                    sem_o.at[s_r])

        for s_r in range(S_REM):
            wait_oblock(s_r)
        for s_r in range(S_REM, 8):
            wait_oblock(s_r)

    return gather_kernel(x_p, table_p)


def kernel(x, table):
    x_p = jnp.transpose(x.astype(jnp.int32), (1, 0))
    tab_f = jnp.reshape(jnp.transpose(table, (1, 0)), (EMBED_DIM * VOCAB,))
    out_p = _encode(x_p, tab_f)
    return jnp.transpose(out_p, (2, 0, 1))


# speedup vs baseline: 1.1927x; 1.1927x over previous
import functools

import jax
import jax.numpy as jnp
from jax import lax
from jax.experimental import pallas as pl
from jax.experimental.pallas import tpu as pltpu
from jax.experimental.pallas import tpu_sc as plsc

VOCAB = 5053
EMBED_DIM = 32
SEQ = 100
BATCH = 16384

NUM_CORES = 2
NUM_SUBCORES = 16
NUM_WORKERS = NUM_CORES * NUM_SUBCORES

HALF_E = EMBED_DIM // 2
TC_PER_WORKER = (BATCH // 128) // (NUM_WORKERS // 2)
S_TILES_FULL = SEQ // 8
S_REM = SEQ - S_TILES_FULL * 8
UNITS_A = S_TILES_FULL * TC_PER_WORKER
VSTRIDE = 5056


def _encode(x_p, table_p):
    mesh = plsc.VectorSubcoreMesh(
        core_axis_name="c", subcore_axis_name="s",
        num_cores=NUM_CORES, num_subcores=NUM_SUBCORES,
    )

    @functools.partial(
        pl.kernel,
        out_type=jax.ShapeDtypeStruct((SEQ, EMBED_DIM, BATCH), jnp.float32),
        mesh=mesh,
        scratch_types=[
            pltpu.VMEM((HALF_E * VOCAB,), jnp.float32),
            pltpu.VMEM((2, 8, 128), jnp.int32),
            pltpu.VMEM((2, 8, HALF_E, 128), jnp.float32),
            pltpu.SemaphoreType.DMA,
            pltpu.SemaphoreType.DMA((2,)),
            pltpu.SemaphoreType.DMA((2, 8)),
        ],
        compiler_params=pltpu.CompilerParams(needs_layout_passes=False),
    )
    def gather_kernel(x_hbm, tab_hbm, out_hbm,
                      tab_v, xbuf, obuf, sem_t, sem_x, sem_o):
        cid = lax.axis_index("c")
        sid = lax.axis_index("s")
        wid = sid * NUM_CORES + cid
        h = wid // (NUM_WORKERS // 2)
        grp = wid % (NUM_WORKERS // 2)
        e_base = h * HALF_E
        tc_base = grp * TC_PER_WORKER

        pltpu.async_copy(
            tab_hbm.at[pl.ds(e_base * VOCAB, HALF_E * VOCAB)], tab_v,
            sem_t).wait()

        def unit_coords(t):
            s_t = t // TC_PER_WORKER
            tc = tc_base + lax.rem(t, TC_PER_WORKER)
            return s_t * 8, tc * 128

        def start_xload(t, p):
            s0, b0 = unit_coords(t)
            pltpu.async_copy(
                x_hbm.at[pl.ds(s0, 8), pl.ds(b0, 128)], xbuf.at[p],
                sem_x.at[p])

        def wait_xload(t, p):
            s0, b0 = unit_coords(t)
            pltpu.make_async_copy(
                x_hbm.at[pl.ds(s0, 8), pl.ds(b0, 128)], xbuf.at[p],
                sem_x.at[p]).wait()

        def wait_oblock(q, s_r):
            pltpu.make_async_copy(
                obuf.at[q, s_r],
                out_hbm.at[0, pl.ds(e_base, HALF_E), pl.ds(0, 128)],
                sem_o.at[q, s_r]).wait()

        def compute_srow(p, s_r):
            vs = [xbuf[p, s_r, pl.ds(g * 16, 16)] for g in range(8)]

            @plsc.parallel_loop(0, HALF_E)
            def erow(e_r):
                base = jnp.full((16,), e_r * VOCAB, jnp.int32)
                for g in range(8):
                    vals = plsc.load_gather(tab_v, [vs[g] + base])
                    obuf[p, s_r, e_r, pl.ds(g * 16, 16)] = vals

        def store_srow(t, q, s_r):
            s0, b0 = unit_coords(t)
            pltpu.async_copy(
                obuf.at[q, s_r],
                out_hbm.at[s0 + s_r, pl.ds(e_base, HALF_E), pl.ds(b0, 128)],
                sem_o.at[q, s_r])

        start_xload(0, 0)

        def outer(tp, carry):
            for p in range(2):
                t = tp * 2 + p
                wait_xload(t, p)

                @pl.when(t < UNITS_A - 1)
                def _():
                    start_xload(t + 1, 1 - p)

                for s_r in range(8):
                    @pl.when(t >= 2)
                    def _():
                        wait_oblock(p, s_r)
                    compute_srow(p, s_r)
                    store_srow(t, p, s_r)
            return carry

        lax.fori_loop(0, UNITS_A // 2, outer, 0)

        for j in range(TC_PER_WORKER):
            q = j % 2
            b0 = (tc_base + j) * 128
            pltpu.async_copy(
                x_hbm.at[pl.ds(S_TILES_FULL * 8, S_REM), pl.ds(b0, 128)],
                xbuf.at[q, pl.ds(0, S_REM)], sem_x.at[q]).wait()
            for s_r in range(S_REM):
                wait_oblock(q, s_r)
                compute_srow(q, s_r)
                pltpu.async_copy(
                    obuf.at[q, s_r],
                    out_hbm.at[S_TILES_FULL * 8 + s_r,
                               pl.ds(e_base, HALF_E), pl.ds(b0, 128)],
                    sem_o.at[q, s_r])

        for q in range(2):
            for s_r in range(8):
                wait_oblock(q, s_r)

    return gather_kernel(x_p, table_p)


def kernel(x, table):
    x_p = jnp.transpose(x.astype(jnp.int32), (1, 0))
    tab_f = jnp.reshape(jnp.transpose(table, (1, 0)), (EMBED_DIM * VOCAB,))
    out_p = _encode(x_p, tab_f)
    return jnp.transpose(out_p, (2, 0, 1))


# speedup vs baseline: 1.2086x; 1.0133x over previous
import functools

import jax
import jax.numpy as jnp
from jax import lax
from jax.experimental import pallas as pl
from jax.experimental.pallas import tpu as pltpu
from jax.experimental.pallas import tpu_sc as plsc

VOCAB = 5053
EMBED_DIM = 32
SEQ = 100
BATCH = 16384

NUM_CORES = 2
NUM_SUBCORES = 16
NUM_WORKERS = NUM_CORES * NUM_SUBCORES

HALF_E = EMBED_DIM // 2
TC_PER_WORKER = (BATCH // 128) // (NUM_WORKERS // 2)
S_TILES_FULL = SEQ // 8
S_REM = SEQ - S_TILES_FULL * 8
UNITS_A = S_TILES_FULL * TC_PER_WORKER
VSTRIDE = 5056


def _encode(x_p, table_p):
    mesh = plsc.VectorSubcoreMesh(
        core_axis_name="c", subcore_axis_name="s",
        num_cores=NUM_CORES, num_subcores=NUM_SUBCORES,
    )

    @functools.partial(
        pl.kernel,
        out_type=jax.ShapeDtypeStruct((SEQ, EMBED_DIM, BATCH), jnp.float32),
        mesh=mesh,
        scratch_types=[
            pltpu.VMEM((HALF_E * VOCAB,), jnp.float32),
            pltpu.VMEM((2, 8, 128), jnp.int32),
            pltpu.VMEM((2, 8, HALF_E, 128), jnp.float32),
            pltpu.VMEM_SHARED((EMBED_DIM * VOCAB,), jnp.float32),
            pltpu.SemaphoreType.DMA,
            pltpu.SemaphoreType.DMA((2,)),
            pltpu.SemaphoreType.DMA((2, 8)),
        ],
        compiler_params=pltpu.CompilerParams(needs_layout_passes=False),
    )
    def gather_kernel(x_hbm, tab_hbm, out_hbm,
                      tab_v, xbuf, obuf, tab_sh, sem_t, sem_x, sem_o):
        cid = lax.axis_index("c")
        sid = lax.axis_index("s")
        wid = sid * NUM_CORES + cid
        h = wid // (NUM_WORKERS // 2)
        grp = wid % (NUM_WORKERS // 2)
        e_base = h * HALF_E
        tc_base = grp * TC_PER_WORKER

        @pl.when(sid == 0)
        def _():
            pltpu.async_copy(tab_hbm, tab_sh, sem_t).wait()

        plsc.subcore_barrier()
        pltpu.async_copy(
            tab_sh.at[pl.ds(e_base * VOCAB, HALF_E * VOCAB)], tab_v,
            sem_t).wait()

        def unit_coords(t):
            s_t = t // TC_PER_WORKER
            tc = tc_base + lax.rem(t, TC_PER_WORKER)
            return s_t * 8, tc * 128

        def start_xload(t, p):
            s0, b0 = unit_coords(t)
            pltpu.async_copy(
                x_hbm.at[pl.ds(s0, 8), pl.ds(b0, 128)], xbuf.at[p],
                sem_x.at[p])

        def wait_xload(t, p):
            s0, b0 = unit_coords(t)
            pltpu.make_async_copy(
                x_hbm.at[pl.ds(s0, 8), pl.ds(b0, 128)], xbuf.at[p],
                sem_x.at[p]).wait()

        def wait_oblock(q, s_r):
            pltpu.make_async_copy(
                obuf.at[q, s_r],
                out_hbm.at[0, pl.ds(e_base, HALF_E), pl.ds(0, 128)],
                sem_o.at[q, s_r]).wait()

        def compute_srow(p, s_r):
            vs = [xbuf[p, s_r, pl.ds(g * 16, 16)] for g in range(8)]

            @plsc.parallel_loop(0, HALF_E)
            def erow(e_r):
                base = jnp.full((16,), e_r * VOCAB, jnp.int32)
                for g in range(8):
                    vals = plsc.load_gather(tab_v, [vs[g] + base])
                    obuf[p, s_r, e_r, pl.ds(g * 16, 16)] = vals

        def store_srow(t, q, s_r):
            s0, b0 = unit_coords(t)
            pltpu.async_copy(
                obuf.at[q, s_r],
                out_hbm.at[s0 + s_r, pl.ds(e_base, HALF_E), pl.ds(b0, 128)],
                sem_o.at[q, s_r])

        start_xload(0, 0)

        def outer(tp, carry):
            for p in range(2):
                t = tp * 2 + p
                wait_xload(t, p)

                @pl.when(t < UNITS_A - 1)
                def _():
                    start_xload(t + 1, 1 - p)

                for s_r in range(8):
                    @pl.when(t >= 2)
                    def _():
                        wait_oblock(p, s_r)
                    compute_srow(p, s_r)
                    store_srow(t, p, s_r)
            return carry

        lax.fori_loop(0, UNITS_A // 2, outer, 0)

        for j in range(TC_PER_WORKER):
            q = j % 2
            b0 = (tc_base + j) * 128
            pltpu.async_copy(
                x_hbm.at[pl.ds(S_TILES_FULL * 8, S_REM), pl.ds(b0, 128)],
                xbuf.at[q, pl.ds(0, S_REM)], sem_x.at[q]).wait()
            for s_r in range(S_REM):
                wait_oblock(q, s_r)
                compute_srow(q, s_r)
                pltpu.async_copy(
                    obuf.at[q, s_r],
                    out_hbm.at[S_TILES_FULL * 8 + s_r,
                               pl.ds(e_base, HALF_E), pl.ds(b0, 128)],
                    sem_o.at[q, s_r])

        for q in range(2):
            for s_r in range(8):
                wait_oblock(q, s_r)

    return gather_kernel(x_p, table_p)


def kernel(x, table):
    x_p = jnp.transpose(x.astype(jnp.int32), (1, 0))
    tab_f = jnp.reshape(jnp.transpose(table, (1, 0)), (EMBED_DIM * VOCAB,))
    out_p = _encode(x_p, tab_f)
    return jnp.transpose(out_p, (2, 0, 1))


# speedup vs baseline: 1.2088x; 1.0002x over previous
import functools

import jax
import jax.numpy as jnp
from jax import lax
from jax.experimental import pallas as pl
from jax.experimental.pallas import tpu as pltpu
from jax.experimental.pallas import tpu_sc as plsc

VOCAB = 5053
EMBED_DIM = 32
SEQ = 100
BATCH = 16384

NUM_CORES = 2
NUM_SUBCORES = 16
NUM_WORKERS = NUM_CORES * NUM_SUBCORES

HALF_E = EMBED_DIM // 2
TC_PER_WORKER = (BATCH // 128) // (NUM_WORKERS // 2)
S_TILES_FULL = SEQ // 8
S_REM = SEQ - S_TILES_FULL * 8
UNITS_A = S_TILES_FULL * TC_PER_WORKER
VSTRIDE = 5056


def _encode(x_p, table_p):
    mesh = plsc.VectorSubcoreMesh(
        core_axis_name="c", subcore_axis_name="s",
        num_cores=NUM_CORES, num_subcores=NUM_SUBCORES,
    )

    @functools.partial(
        pl.kernel,
        out_type=jax.ShapeDtypeStruct((SEQ, EMBED_DIM, BATCH), jnp.float32),
        mesh=mesh,
        scratch_types=[
            pltpu.VMEM((HALF_E * VOCAB,), jnp.float32),
            pltpu.VMEM((2, 8, 128), jnp.int32),
            pltpu.VMEM((2, 8, HALF_E, 128), jnp.float32),
            pltpu.VMEM_SHARED((EMBED_DIM * VOCAB,), jnp.float32),
            pltpu.SemaphoreType.DMA,
            pltpu.SemaphoreType.DMA((2,)),
            pltpu.SemaphoreType.DMA((2, 8)),
        ],
        compiler_params=pltpu.CompilerParams(needs_layout_passes=False),
    )
    def gather_kernel(x_hbm, tab_hbm, out_hbm,
                      tab_v, xbuf, obuf, tab_sh, sem_t, sem_x, sem_o):
        cid = lax.axis_index("c")
        sid = lax.axis_index("s")
        wid = sid * NUM_CORES + cid
        h = wid // (NUM_WORKERS // 2)
        grp = wid % (NUM_WORKERS // 2)
        e_base = h * HALF_E
        tc_base = grp * TC_PER_WORKER

        def unit_coords(t):
            s_t = t // TC_PER_WORKER
            tc = tc_base + lax.rem(t, TC_PER_WORKER)
            return s_t * 8, tc * 128

        def start_xload(t, p):
            s0, b0 = unit_coords(t)
            pltpu.async_copy(
                x_hbm.at[pl.ds(s0, 8), pl.ds(b0, 128)], xbuf.at[p],
                sem_x.at[p])

        def wait_xload(t, p):
            s0, b0 = unit_coords(t)
            pltpu.make_async_copy(
                x_hbm.at[pl.ds(s0, 8), pl.ds(b0, 128)], xbuf.at[p],
                sem_x.at[p]).wait()

        def wait_oblock(q, s_r):
            pltpu.make_async_copy(
                obuf.at[q, s_r],
                out_hbm.at[0, pl.ds(e_base, HALF_E), pl.ds(0, 128)],
                sem_o.at[q, s_r]).wait()

        def compute_srow(p, s_r):
            vs = [xbuf[p, s_r, pl.ds(g * 16, 16)] for g in range(8)]

            @plsc.parallel_loop(0, HALF_E)
            def erow(e_r):
                base = jnp.full((16,), e_r * VOCAB, jnp.int32)
                for g in range(8):
                    vals = plsc.load_gather(tab_v, [vs[g] + base])
                    obuf[p, s_r, e_r, pl.ds(g * 16, 16)] = vals

        def store_srow(t, q, s_r):
            s0, b0 = unit_coords(t)
            pltpu.async_copy(
                obuf.at[q, s_r],
                out_hbm.at[s0 + s_r, pl.ds(e_base, HALF_E), pl.ds(b0, 128)],
                sem_o.at[q, s_r])

        start_xload(0, 0)

        @pl.when(sid == 0)
        def _():
            pltpu.async_copy(tab_hbm, tab_sh, sem_t).wait()

        plsc.subcore_barrier()
        pltpu.async_copy(
            tab_sh.at[pl.ds(e_base * VOCAB, HALF_E * VOCAB)], tab_v,
            sem_t).wait()


        def outer(tp, carry):
            for p in range(2):
                t = tp * 2 + p
                wait_xload(t, p)

                @pl.when(t < UNITS_A - 1)
                def _():
                    start_xload(t + 1, 1 - p)

                for s_r in range(8):
                    @pl.when(t >= 2)
                    def _():
                        wait_oblock(p, s_r)
                    compute_srow(p, s_r)
                    store_srow(t, p, s_r)
            return carry

        lax.fori_loop(0, UNITS_A // 2, outer, 0)

        for j in range(TC_PER_WORKER):
            q = j % 2
            b0 = (tc_base + j) * 128
            pltpu.async_copy(
                x_hbm.at[pl.ds(S_TILES_FULL * 8, S_REM), pl.ds(b0, 128)],
                xbuf.at[q, pl.ds(0, S_REM)], sem_x.at[q]).wait()
            for s_r in range(S_REM):
                wait_oblock(q, s_r)
                compute_srow(q, s_r)
                pltpu.async_copy(
                    obuf.at[q, s_r],
                    out_hbm.at[S_TILES_FULL * 8 + s_r,
                               pl.ds(e_base, HALF_E), pl.ds(b0, 128)],
                    sem_o.at[q, s_r])

        for q in range(2):
            for s_r in range(8):
                wait_oblock(q, s_r)

    return gather_kernel(x_p, table_p)


def kernel(x, table):
    x_p = jnp.transpose(x.astype(jnp.int32), (1, 0))
    tab_f = jnp.reshape(jnp.transpose(table, (1, 0)), (EMBED_DIM * VOCAB,))
    out_p = _encode(x_p, tab_f)
    return jnp.transpose(out_p, (2, 0, 1))
